# Initial kernel scaffold; baseline (speedup 1.0000x reference)
import jax
import jax.numpy as jnp
from jax.experimental import pallas as pl


def kernel(X, X_ref, labels_ref):
    raise NotImplementedError("write your pallas kernel here")


# speedup vs baseline: 2.2153x; 2.2153x over previous
import functools

import jax
import jax.numpy as jnp
import numpy as np
from jax.experimental import pallas as pl
from jax.experimental.pallas import tpu as pltpu

K_NEIGH = 3
N_CLASSES = 20
NEG = -1.0e30

_EPS = np.float64(np.finfo(np.float64).eps)


def _logit_table():
    c = np.arange(K_NEIGH + 1, dtype=np.float32)
    p = c / np.float32(K_NEIGH)
    one = np.float32(1.0)
    eps = np.float32(_EPS)
    tab = np.log((p / (one - p + eps) + eps).astype(np.float32)).astype(np.float32)
    tab[K_NEIGH] = np.float32(np.inf)
    return tab


_LOGITS = _logit_table()


def _fused_knn_kernel(x_ref, xr_ref, lab_ref, o_ref, rv_ref, rl_ref, *,
                      blk, n_rows, n_blocks, n_classes):
    j = pl.program_id(0)
    q = x_ref.shape[0]

    @pl.when(j == 0)
    def _init():
        rv_ref[...] = jnp.full((q, 8), NEG, jnp.float32)
        rl_ref[...] = jnp.zeros((q, 8), jnp.int32)

    x = x_ref[...]
    xr = xr_ref[...]

    sims = jax.lax.dot_general(x, xr, (((1,), (1,)), ((), ())),
                               preferred_element_type=jnp.float32)

    col = jax.lax.broadcasted_iota(jnp.int32, (1, blk), 1) + j * blk
    bias = jnp.where(col < n_rows, 0.0, NEG).astype(jnp.float32)
    v = sims + bias

    bl = lab_ref[0]

    ms, ls = [], []
    for _ in range(K_NEIGH):
        m = jnp.max(v, axis=1, keepdims=True)
        eq = v == m
        lab = jnp.max(jnp.where(eq, bl, -1), axis=1, keepdims=True)
        v = jnp.where(eq, NEG, v)
        ms.append(m)
        ls.append(lab)

    cand_v = jnp.concatenate([rv_ref[:, 0:K_NEIGH]] + ms, axis=1)
    cand_l = jnp.concatenate([rl_ref[:, 0:K_NEIGH]] + ls, axis=1)
    w = cand_v
    nv, nl = [], []
    for _ in range(K_NEIGH):
        m = jnp.max(w, axis=1, keepdims=True)
        eq = w == m
        lab = jnp.max(jnp.where(eq, cand_l, -1), axis=1, keepdims=True)
        w = jnp.where(eq, NEG, w)
        nv.append(m)
        nl.append(lab)
    rv_ref[:, 0:K_NEIGH] = jnp.concatenate(nv, axis=1)
    rl_ref[:, 0:K_NEIGH] = jnp.concatenate(nl, axis=1)

    @pl.when(j == n_blocks - 1)
    def _finish():
        cls = jax.lax.broadcasted_iota(jnp.int32, (q, n_classes), 1)
        counts = jnp.zeros((q, n_classes), jnp.int32)
        for t in range(K_NEIGH):
            counts = counts + (nl[t] == cls).astype(jnp.int32)
        out = jnp.full((q, n_classes), _LOGITS[0], jnp.float32)
        for c in range(1, K_NEIGH + 1):
            out = jnp.where(counts == c, jnp.float32(_LOGITS[c]), out)
        o_ref[...] = out


def kernel(X, X_ref, labels_ref):
    q, d = X.shape
    r = X_ref.shape[0]
    blk = 2048
    n_blocks = -(-r // blk)
    r_pad = n_blocks * blk

    x_norm = jnp.maximum(jnp.linalg.norm(X, axis=1, keepdims=True), 1e-12)
    r_norm = jnp.maximum(jnp.linalg.norm(X_ref, axis=1, keepdims=True), 1e-12)
    Xn = X / x_norm
    Rn = X_ref / r_norm

    Rn = jnp.pad(Rn, ((0, r_pad - r), (0, 0)))
    lab = jnp.pad(labels_ref.astype(jnp.int32), (0, r_pad - r))
    lab = lab.reshape(n_blocks, 1, blk)

    body = functools.partial(_fused_knn_kernel, blk=blk, n_rows=r,
                             n_blocks=n_blocks, n_classes=N_CLASSES)
    return pl.pallas_call(
        body,
        grid=(n_blocks,),
        in_specs=[
            pl.BlockSpec((q, d), lambda j: (0, 0)),
            pl.BlockSpec((blk, d), lambda j: (j, 0)),
            pl.BlockSpec((1, 1, blk), lambda j: (j, 0, 0)),
        ],
        out_specs=pl.BlockSpec((q, N_CLASSES), lambda j: (0, 0)),
        out_shape=jax.ShapeDtypeStruct((q, N_CLASSES), jnp.float32),
        scratch_shapes=[
            pltpu.VMEM((q, 8), jnp.float32),
            pltpu.VMEM((q, 8), jnp.int32),
        ],
    )(Xn, Rn, lab)


# speedup vs baseline: 2.2288x; 1.0061x over previous
import functools

import jax
import jax.numpy as jnp
import numpy as np
from jax.experimental import pallas as pl
from jax.experimental.pallas import tpu as pltpu

K_NEIGH = 3
N_CLASSES = 20
NEG = -1.0e30
PAD_VAL = -1.0e30


def _logit_table():
    c = np.arange(K_NEIGH + 1, dtype=np.float32)
    p = c / np.float32(K_NEIGH)
    one = np.float32(1.0)
    eps = np.float32(np.finfo(np.float64).eps)
    tab = np.log((p / (one - p + eps) + eps).astype(np.float32)).astype(np.float32)
    tab[K_NEIGH] = np.float32(np.inf)
    return tab


_LOGITS = _logit_table()


def _knn_kernel(x_ref, xr_ref, oh_ref, o_ref, rv_ref, *,
                blk, n_blocks, n_classes):
    i = pl.program_id(0)
    j = pl.program_id(1)
    q = x_ref.shape[0]

    x = x_ref[...]
    xr = xr_ref[...]
    sims = jax.lax.dot_general(x, xr, (((1,), (1,)), ((), ())),
                               preferred_element_type=jnp.float32)

    @pl.when(i == 0)
    def _phase1():
        @pl.when(j == 0)
        def _init():
            rv_ref[...] = jnp.full((q, 8), NEG, jnp.float32)

        v = sims
        ms = []
        for t in range(K_NEIGH):
            m = jnp.max(v, axis=1, keepdims=True)
            if t < K_NEIGH - 1:
                v = jnp.where(v == m, NEG, v)
            ms.append(m)

        w = jnp.concatenate([rv_ref[:, 0:K_NEIGH]] + ms, axis=1)
        nv = []
        for t in range(K_NEIGH):
            m = jnp.max(w, axis=1, keepdims=True)
            if t < K_NEIGH - 1:
                w = jnp.where(w == m, NEG, w)
            nv.append(m)
        rv_ref[:, 0:K_NEIGH] = jnp.concatenate(nv, axis=1)

    @pl.when(i == 1)
    def _phase2():
        thr = rv_ref[:, K_NEIGH - 1:K_NEIGH]
        maskf = jnp.where(sims >= thr, 1.0, 0.0).astype(jnp.float32)
        c = jax.lax.dot_general(maskf, oh_ref[...], (((1,), (0,)), ((), ())),
                                preferred_element_type=jnp.float32)

        @pl.when(j == 0)
        def _set():
            o_ref[...] = c

        @pl.when(j > 0)
        def _acc():
            o_ref[...] = o_ref[...] + c

        @pl.when(j == n_blocks - 1)
        def _finish():
            counts = o_ref[...]
            out = jnp.full((q, n_classes), _LOGITS[0], jnp.float32)
            for cnt in range(1, K_NEIGH + 1):
                out = jnp.where(counts == np.float32(cnt),
                                jnp.float32(_LOGITS[cnt]), out)
            o_ref[...] = out


def kernel(X, X_ref, labels_ref):
    q, d = X.shape
    r = X_ref.shape[0]
    blk = 2048
    n_blocks = -(-r // blk)
    r_pad = n_blocks * blk

    x_norm = jnp.maximum(jnp.linalg.norm(X, axis=1, keepdims=True), 1e-12)
    r_norm = jnp.maximum(jnp.linalg.norm(X_ref, axis=1, keepdims=True), 1e-12)
    Xn = X / x_norm
    Rn = X_ref / r_norm

    Rn = jnp.pad(Rn, ((0, r_pad - r), (0, 0)), constant_values=PAD_VAL)
    lab = labels_ref.astype(jnp.int32)
    onehot = (lab[:, None] ==
              jnp.arange(N_CLASSES, dtype=jnp.int32)[None, :]).astype(jnp.float32)
    onehot = jnp.pad(onehot, ((0, r_pad - r), (0, 0)))

    body = functools.partial(_knn_kernel, blk=blk,
                             n_blocks=n_blocks, n_classes=N_CLASSES)
    return pl.pallas_call(
        body,
        grid=(2, n_blocks),
        in_specs=[
            pl.BlockSpec((q, d), lambda i, j: (0, 0)),
            pl.BlockSpec((blk, d), lambda i, j: (j, 0)),
            pl.BlockSpec((blk, N_CLASSES), lambda i, j: (j, 0)),
        ],
        out_specs=pl.BlockSpec((q, N_CLASSES), lambda i, j: (0, 0)),
        out_shape=jax.ShapeDtypeStruct((q, N_CLASSES), jnp.float32),
        scratch_shapes=[
            pltpu.VMEM((q, 8), jnp.float32),
        ],
    )(Xn, Rn, onehot)


# speedup vs baseline: 2.5004x; 1.1219x over previous
import functools

import jax
import jax.numpy as jnp
import numpy as np
from jax.experimental import pallas as pl
from jax.experimental.pallas import tpu as pltpu

K_NEIGH = 3
N_CLASSES = 20
NEG = -1.0e30
PAD_VAL = -1.0e30


def _logit_table():
    c = np.arange(K_NEIGH + 1, dtype=np.float32)
    p = c / np.float32(K_NEIGH)
    one = np.float32(1.0)
    eps = np.float32(np.finfo(np.float64).eps)
    tab = np.log((p / (one - p + eps) + eps).astype(np.float32)).astype(np.float32)
    tab[K_NEIGH] = np.float32(np.inf)
    return tab


_LOGITS = _logit_table()


def _knn_kernel(x_ref, xr_ref, lab_ref, o_ref, rv_ref, *,
                blk, n_blocks, n_classes):
    i = pl.program_id(0)
    j = pl.program_id(1)
    q = x_ref.shape[0]

    x = x_ref[...]
    xr = xr_ref[...]
    sims = jax.lax.dot_general(x, xr, (((1,), (1,)), ((), ())),
                               preferred_element_type=jnp.float32)

    @pl.when(i == 0)
    def _phase1():
        @pl.when(j == 0)
        def _init():
            rv_ref[...] = jnp.full((q, 16), NEG, jnp.float32)

        cands = []
        p = sims
        while p.shape[1] > 128:
            h = p.shape[1] // 2
            left = p[:, :h]
            right = p[:, h:]
            qmin = jnp.minimum(left, right)
            p = jnp.maximum(left, right)
            cands.append(jnp.max(qmin, axis=1, keepdims=True))

        for t in range(K_NEIGH):
            m = jnp.max(p, axis=1, keepdims=True)
            if t < K_NEIGH - 1:
                p = jnp.where(p == m, NEG, p)
            cands.append(m)

        w = jnp.concatenate([rv_ref[:, 0:K_NEIGH]] + cands, axis=1)
        nv = []
        for t in range(K_NEIGH):
            m = jnp.max(w, axis=1, keepdims=True)
            if t < K_NEIGH - 1:
                w = jnp.where(w == m, NEG, w)
            nv.append(m)
        rv_ref[:, 0:K_NEIGH] = jnp.concatenate(nv, axis=1)

    @pl.when(i == 1)
    def _phase2():
        thr = rv_ref[:, K_NEIGH - 1:K_NEIGH]
        maskf = jnp.where(sims >= thr, 1.0, 0.0).astype(jnp.float32)
        lab = lab_ref[...]
        cls = jax.lax.broadcasted_iota(jnp.int32, (1, n_classes), 1)
        oh = (lab == cls).astype(jnp.float32)
        c = jax.lax.dot_general(maskf, oh, (((1,), (0,)), ((), ())),
                                preferred_element_type=jnp.float32)

        @pl.when(j == 0)
        def _set():
            o_ref[...] = c

        @pl.when(j > 0)
        def _acc():
            o_ref[...] = o_ref[...] + c

        @pl.when(j == n_blocks - 1)
        def _finish():
            counts = o_ref[...]
            out = jnp.full((q, n_classes), _LOGITS[0], jnp.float32)
            for cnt in range(1, K_NEIGH + 1):
                out = jnp.where(counts == np.float32(cnt),
                                jnp.float32(_LOGITS[cnt]), out)
            o_ref[...] = out


def kernel(X, X_ref, labels_ref):
    q, d = X.shape
    r = X_ref.shape[0]
    blk = 2048
    n_blocks = -(-r // blk)
    r_pad = n_blocks * blk

    x_norm = jnp.maximum(jnp.linalg.norm(X, axis=1, keepdims=True), 1e-12)
    r_norm = jnp.maximum(jnp.linalg.norm(X_ref, axis=1, keepdims=True), 1e-12)
    Xn = X / x_norm
    Rn = jnp.pad(X_ref / r_norm, ((0, r_pad - r), (0, 0)),
                 constant_values=PAD_VAL)
    lab = jnp.pad(labels_ref.astype(jnp.int32), (0, r_pad - r),
                  constant_values=-1).reshape(r_pad, 1)

    body = functools.partial(_knn_kernel, blk=blk,
                             n_blocks=n_blocks, n_classes=N_CLASSES)
    return pl.pallas_call(
        body,
        grid=(2, n_blocks),
        in_specs=[
            pl.BlockSpec((q, d), lambda i, j: (0, 0)),
            pl.BlockSpec((blk, d), lambda i, j: (j, 0)),
            pl.BlockSpec((blk, 1), lambda i, j: (j, 0)),
        ],
        out_specs=pl.BlockSpec((q, N_CLASSES), lambda i, j: (0, 0)),
        out_shape=jax.ShapeDtypeStruct((q, N_CLASSES), jnp.float32),
        scratch_shapes=[
            pltpu.VMEM((q, 16), jnp.float32),
        ],
    )(Xn, Rn, lab)


# speedup vs baseline: 2.7183x; 1.0871x over previous
import functools

import jax
import jax.numpy as jnp
import numpy as np
from jax.experimental import pallas as pl
from jax.experimental.pallas import tpu as pltpu

K_NEIGH = 3
N_CLASSES = 20
NEG = -1.0e30
PAD_VAL = -1.0e30


def _logit_table():
    c = np.arange(K_NEIGH + 1, dtype=np.float32)
    p = c / np.float32(K_NEIGH)
    one = np.float32(1.0)
    eps = np.float32(np.finfo(np.float64).eps)
    tab = np.log((p / (one - p + eps) + eps).astype(np.float32)).astype(np.float32)
    tab[K_NEIGH] = np.float32(np.inf)
    return tab


_LOGITS = _logit_table()


def _knn_kernel(x_ref, xr_ref, lab_ref, o_ref, rv_ref, *,
                blk, n_blocks, n_classes):
    i = pl.program_id(0)
    j = pl.program_id(1)
    q = x_ref.shape[0]

    x = x_ref[...]
    xr = xr_ref[...]
    sims = jax.lax.dot_general(x, xr, (((1,), (1,)), ((), ())),
                               preferred_element_type=jnp.float32)

    @pl.when(i == 0)
    def _phase1():
        @pl.when(j == 0)
        def _init():
            rv_ref[...] = jnp.full((q, 16), NEG, jnp.float32)

        cands = []
        p = sims
        while p.shape[1] > 128:
            h = p.shape[1] // 2
            left = p[:, :h]
            right = p[:, h:]
            qmin = jnp.minimum(left, right)
            p = jnp.maximum(left, right)
            cands.append(jnp.max(qmin, axis=1, keepdims=True))

        for t in range(K_NEIGH):
            m = jnp.max(p, axis=1, keepdims=True)
            if t < K_NEIGH - 1:
                p = jnp.where(p == m, NEG, p)
            cands.append(m)

        w = jnp.concatenate([rv_ref[:, 0:K_NEIGH]] + cands, axis=1)
        nv = []
        for t in range(K_NEIGH):
            m = jnp.max(w, axis=1, keepdims=True)
            if t < K_NEIGH - 1:
                w = jnp.where(w == m, NEG, w)
            nv.append(m)
        rv_ref[:, 0:K_NEIGH] = jnp.concatenate(nv, axis=1)

    @pl.when(i == 1)
    def _phase2():
        thr = rv_ref[:, K_NEIGH - 1:K_NEIGH]
        maskf = jnp.where(sims >= thr, 1.0, 0.0).astype(jnp.float32)
        lab = lab_ref[...]
        cls = jax.lax.broadcasted_iota(jnp.int32, (1, n_classes), 1)
        oh = (lab == cls).astype(jnp.float32)
        c = jax.lax.dot_general(maskf, oh, (((1,), (0,)), ((), ())),
                                preferred_element_type=jnp.float32)

        @pl.when(j == 0)
        def _set():
            o_ref[...] = c

        @pl.when(j > 0)
        def _acc():
            o_ref[...] = o_ref[...] + c

        @pl.when(j == n_blocks - 1)
        def _finish():
            counts = o_ref[...]
            out = jnp.full((q, n_classes), _LOGITS[0], jnp.float32)
            for cnt in range(1, K_NEIGH + 1):
                out = jnp.where(counts == np.float32(cnt),
                                jnp.float32(_LOGITS[cnt]), out)
            o_ref[...] = out


def kernel(X, X_ref, labels_ref):
    q, d = X.shape
    r = X_ref.shape[0]
    blk = 4096
    n_blocks = -(-r // blk)
    r_pad = n_blocks * blk

    x_norm = jnp.maximum(jnp.linalg.norm(X, axis=1, keepdims=True), 1e-12)
    r_norm = jnp.maximum(jnp.linalg.norm(X_ref, axis=1, keepdims=True), 1e-12)
    Xn = X / x_norm
    Rn = jnp.pad(X_ref / r_norm, ((0, r_pad - r), (0, 0)),
                 constant_values=PAD_VAL)
    lab = jnp.pad(labels_ref.astype(jnp.int32), (0, r_pad - r),
                  constant_values=-1).reshape(r_pad, 1)

    body = functools.partial(_knn_kernel, blk=blk,
                             n_blocks=n_blocks, n_classes=N_CLASSES)
    return pl.pallas_call(
        body,
        grid=(2, n_blocks),
        in_specs=[
            pl.BlockSpec((q, d), lambda i, j: (0, 0)),
            pl.BlockSpec((blk, d), lambda i, j: (j, 0)),
            pl.BlockSpec((blk, 1), lambda i, j: (j, 0)),
        ],
        out_specs=pl.BlockSpec((q, N_CLASSES), lambda i, j: (0, 0)),
        out_shape=jax.ShapeDtypeStruct((q, N_CLASSES), jnp.float32),
        scratch_shapes=[
            pltpu.VMEM((q, 16), jnp.float32),
        ],
    )(Xn, Rn, lab)


# speedup vs baseline: 2.7761x; 1.0213x over previous
import functools

import jax
import jax.numpy as jnp
import numpy as np
from jax.experimental import pallas as pl
from jax.experimental.pallas import tpu as pltpu

K_NEIGH = 3
N_CLASSES = 20
NEG = -1.0e30
PAD_VAL = -1.0e30


def _logit_table():
    c = np.arange(K_NEIGH + 1, dtype=np.float32)
    p = c / np.float32(K_NEIGH)
    one = np.float32(1.0)
    eps = np.float32(np.finfo(np.float64).eps)
    tab = np.log((p / (one - p + eps) + eps).astype(np.float32)).astype(np.float32)
    tab[K_NEIGH] = np.float32(np.inf)
    return tab


_LOGITS = _logit_table()


def _knn_kernel(x_ref, xr_ref, rn_ref, tail_ref, lab_ref, o_ref, rv_ref, *,
                blk, n_blocks, n_classes):
    i = pl.program_id(0)
    j = pl.program_id(1)
    q = x_ref.shape[0]

    x = x_ref[...]
    xr_n = xr_ref[...] / rn_ref[...]
    xr = jnp.where(j == n_blocks - 1, tail_ref[...], xr_n)
    sims = jax.lax.dot_general(x, xr, (((1,), (1,)), ((), ())),
                               preferred_element_type=jnp.float32)

    @pl.when(i == 0)
    def _phase1():
        @pl.when(j == 0)
        def _init():
            rv_ref[...] = jnp.full((q, 16), NEG, jnp.float32)

        cands = []
        p = sims
        while p.shape[1] > 128:
            h = p.shape[1] // 2
            left = p[:, :h]
            right = p[:, h:]
            qmin = jnp.minimum(left, right)
            p = jnp.maximum(left, right)
            cands.append(jnp.max(qmin, axis=1, keepdims=True))

        for t in range(K_NEIGH):
            m = jnp.max(p, axis=1, keepdims=True)
            if t < K_NEIGH - 1:
                p = jnp.where(p == m, NEG, p)
            cands.append(m)

        w = jnp.concatenate([rv_ref[:, 0:K_NEIGH]] + cands, axis=1)
        nv = []
        for t in range(K_NEIGH):
            m = jnp.max(w, axis=1, keepdims=True)
            if t < K_NEIGH - 1:
                w = jnp.where(w == m, NEG, w)
            nv.append(m)
        rv_ref[:, 0:K_NEIGH] = jnp.concatenate(nv, axis=1)

    @pl.when(i == 1)
    def _phase2():
        thr = rv_ref[:, K_NEIGH - 1:K_NEIGH]
        maskf = jnp.where(sims >= thr, 1.0, 0.0).astype(jnp.float32)
        lab = lab_ref[...]
        cls = jax.lax.broadcasted_iota(jnp.int32, (1, n_classes), 1)
        oh = (lab == cls).astype(jnp.float32)
        c = jax.lax.dot_general(maskf, oh, (((1,), (0,)), ((), ())),
                                preferred_element_type=jnp.float32)

        @pl.when(j == 0)
        def _set():
            o_ref[...] = c

        @pl.when(j > 0)
        def _acc():
            o_ref[...] = o_ref[...] + c

        @pl.when(j == n_blocks - 1)
        def _finish():
            counts = o_ref[...]
            out = jnp.full((q, n_classes), _LOGITS[0], jnp.float32)
            for cnt in range(1, K_NEIGH + 1):
                out = jnp.where(counts == np.float32(cnt),
                                jnp.float32(_LOGITS[cnt]), out)
            o_ref[...] = out


def kernel(X, X_ref, labels_ref):
    q, d = X.shape
    r = X_ref.shape[0]
    blk = 4096
    n_blocks = -(-r // blk)
    n_full = r // blk
    r_pad = n_blocks * blk
    tail_lo = n_full * blk

    x_norm = jnp.maximum(jnp.linalg.norm(X, axis=1, keepdims=True), 1e-12)
    r_norm = jnp.maximum(jnp.linalg.norm(X_ref, axis=1, keepdims=True), 1e-12)
    Xn = X / x_norm
    rn = jnp.pad(r_norm, ((0, r_pad - r), (0, 0)), constant_values=1.0)
    tail = jnp.pad(X_ref[tail_lo:] / r_norm[tail_lo:],
                   ((0, r_pad - r), (0, 0)), constant_values=PAD_VAL)
    lab = jnp.pad(labels_ref.astype(jnp.int32), (0, r_pad - r),
                  constant_values=-1).reshape(r_pad, 1)

    last_full = max(n_full - 1, 0)
    body = functools.partial(_knn_kernel, blk=blk,
                             n_blocks=n_blocks, n_classes=N_CLASSES)
    return pl.pallas_call(
        body,
        grid=(2, n_blocks),
        in_specs=[
            pl.BlockSpec((q, d), lambda i, j: (0, 0)),
            pl.BlockSpec((blk, d), lambda i, j: (jnp.minimum(j, last_full), 0)),
            pl.BlockSpec((blk, 1), lambda i, j: (jnp.minimum(j, last_full), 0)),
            pl.BlockSpec((blk, d), lambda i, j: (0, 0)),
            pl.BlockSpec((blk, 1), lambda i, j: (j, 0)),
        ],
        out_specs=pl.BlockSpec((q, N_CLASSES), lambda i, j: (0, 0)),
        out_shape=jax.ShapeDtypeStruct((q, N_CLASSES), jnp.float32),
        scratch_shapes=[
            pltpu.VMEM((q, 16), jnp.float32),
        ],
    )(Xn, X_ref, rn, tail, lab)
